# baseline (speedup 1.0000x reference)
import jax
import jax.numpy as jnp
from jax import lax
from jax.experimental import pallas as pl
from jax.experimental.pallas import tpu as pltpu

_BLOCK = 32
_IN_F = 4096
_OUT_F = 4096
_GROUP = 128
_BLOCKS_PER_GROUP = _GROUP // _BLOCK
_TOKEN_TILE = 512


def _spmm_body(rg_ref, cg_ref, x_ref, w_ref, bias_ref, o_ref, xb_ref):
    n_blocks = w_ref.shape[0]
    xb_ref[...] = x_ref[...].astype(jnp.bfloat16)
    o_ref[...] = jnp.broadcast_to(bias_ref[...], o_ref.shape)

    def blk(b, carry):
        cg = cg_ref[b]
        rg = rg_ref[b]
        xs = xb_ref[:, pl.ds(cg * _GROUP, _GROUP)]
        contrib = jnp.dot(xs, w_ref[b], preferred_element_type=jnp.float32)
        o_ref[:, pl.ds(rg * _GROUP, _GROUP)] += contrib
        return carry

    lax.fori_loop(0, n_blocks, blk, 0, unroll=n_blocks)


@jax.jit
def kernel(x, weight_data, block_rows, block_cols, bias):
    n_tokens = x.shape[0]
    n_blocks = weight_data.shape[0]

    rg = (block_rows // _BLOCKS_PER_GROUP).astype(jnp.int32)
    ro = block_rows % _BLOCKS_PER_GROUP
    cg = (block_cols // _BLOCKS_PER_GROUP).astype(jnp.int32)
    co = block_cols % _BLOCKS_PER_GROUP

    wt = jnp.transpose(weight_data, (0, 2, 1))
    slots4 = jnp.arange(_BLOCKS_PER_GROUP, dtype=jnp.int32)
    oh_co = (co[:, None] == slots4).astype(jnp.bfloat16)
    oh_ro = (ro[:, None] == slots4).astype(jnp.bfloat16)
    w_tiles = (wt.astype(jnp.bfloat16)[:, None, :, None, :]
               * oh_co[:, :, None, None, None]
               * oh_ro[:, None, None, :, None])
    w_tiles = w_tiles.reshape(n_blocks, _GROUP, _GROUP)

    bias2d = bias.reshape(1, _OUT_F)
    grid = (n_tokens // _TOKEN_TILE,)

    grid_spec = pltpu.PrefetchScalarGridSpec(
        num_scalar_prefetch=2,
        grid=grid,
        in_specs=[
            pl.BlockSpec((_TOKEN_TILE, _IN_F), lambda t, rg, cg: (t, 0)),
            pl.BlockSpec((n_blocks, _GROUP, _GROUP),
                         lambda t, rg, cg: (0, 0, 0)),
            pl.BlockSpec((1, _OUT_F), lambda t, rg, cg: (0, 0)),
        ],
        out_specs=pl.BlockSpec((_TOKEN_TILE, _OUT_F),
                               lambda t, rg, cg: (t, 0)),
        scratch_shapes=[pltpu.VMEM((_TOKEN_TILE, _IN_F), jnp.bfloat16)],
    )

    return pl.pallas_call(
        _spmm_body,
        grid_spec=grid_spec,
        out_shape=jax.ShapeDtypeStruct((n_tokens, _OUT_F), jnp.float32),
        compiler_params=pltpu.CompilerParams(
            dimension_semantics=("parallel",),
        ),
    )(rg, cg, x, w_tiles, bias2d)


# speedup vs baseline: 1.0217x; 1.0217x over previous
import jax
import jax.numpy as jnp
from jax import lax
from jax.experimental import pallas as pl
from jax.experimental.pallas import tpu as pltpu

_BLOCK = 32
_IN_F = 4096
_OUT_F = 4096
_GROUP = 128
_BLOCKS_PER_GROUP = _GROUP // _BLOCK
_TOKEN_TILE = 512


def _spmm_body(rg_ref, cg_ref, x_ref, w_ref, bias_ref, o_ref, xb_ref):
    n_blocks = w_ref.shape[0]
    xb_ref[...] = x_ref[...].astype(jnp.bfloat16)
    o_ref[...] = jnp.broadcast_to(bias_ref[...], o_ref.shape)

    def blk(b, carry):
        cg = cg_ref[b]
        rg = rg_ref[b]
        xs = xb_ref[:, pl.ds(cg * _GROUP, _GROUP)]
        contrib = jnp.dot(xs, w_ref[b], preferred_element_type=jnp.float32)
        o_ref[:, pl.ds(rg * _GROUP, _GROUP)] += contrib
        return carry

    lax.fori_loop(0, n_blocks, blk, 0, unroll=64)


@jax.jit
def kernel(x, weight_data, block_rows, block_cols, bias):
    n_tokens = x.shape[0]
    n_blocks = weight_data.shape[0]

    rg = (block_rows // _BLOCKS_PER_GROUP).astype(jnp.int32)
    ro = block_rows % _BLOCKS_PER_GROUP
    cg = (block_cols // _BLOCKS_PER_GROUP).astype(jnp.int32)
    co = block_cols % _BLOCKS_PER_GROUP

    wt = jnp.transpose(weight_data, (0, 2, 1))
    slots4 = jnp.arange(_BLOCKS_PER_GROUP, dtype=jnp.int32)
    oh_co = (co[:, None] == slots4).astype(jnp.bfloat16)
    oh_ro = (ro[:, None] == slots4).astype(jnp.bfloat16)
    w_tiles = (wt.astype(jnp.bfloat16)[:, None, :, None, :]
               * oh_co[:, :, None, None, None]
               * oh_ro[:, None, None, :, None])
    w_tiles = w_tiles.reshape(n_blocks, _GROUP, _GROUP)

    bias2d = bias.reshape(1, _OUT_F)
    grid = (n_tokens // _TOKEN_TILE,)

    grid_spec = pltpu.PrefetchScalarGridSpec(
        num_scalar_prefetch=2,
        grid=grid,
        in_specs=[
            pl.BlockSpec((_TOKEN_TILE, _IN_F), lambda t, rg, cg: (t, 0)),
            pl.BlockSpec((n_blocks, _GROUP, _GROUP),
                         lambda t, rg, cg: (0, 0, 0)),
            pl.BlockSpec((1, _OUT_F), lambda t, rg, cg: (0, 0)),
        ],
        out_specs=pl.BlockSpec((_TOKEN_TILE, _OUT_F),
                               lambda t, rg, cg: (t, 0)),
        scratch_shapes=[pltpu.VMEM((_TOKEN_TILE, _IN_F), jnp.bfloat16)],
    )

    return pl.pallas_call(
        _spmm_body,
        grid_spec=grid_spec,
        out_shape=jax.ShapeDtypeStruct((n_tokens, _OUT_F), jnp.float32),
        compiler_params=pltpu.CompilerParams(
            dimension_semantics=("parallel",),
        ),
    )(rg, cg, x, w_tiles, bias2d)
